# baseline (speedup 1.0000x reference)
import functools

import jax
import jax.numpy as jnp
from jax import lax
from jax.experimental import pallas as pl
from jax.experimental.pallas import tpu as pltpu
from jax.experimental.pallas import tpu_sc as plsc

EMBED = 64
MAXSIZE = 520
B = 4096
L = 50

NUM_CH = 6
ROW = NUM_CH * EMBED
TOKENS_PER_B = L * NUM_CH
NC, NS = 2, 16
NW = NC * NS
NB_PER_W = B // NW
N_BPAIRS = NB_PER_W // 2


def _build_table(size_table):
    def body(st_ref, out_ref):
        pos = lax.broadcasted_iota(jnp.int32, (MAXSIZE, EMBED), 0).astype(jnp.float32)
        col = lax.broadcasted_iota(jnp.int32, (MAXSIZE, EMBED), 1)
        j = (col % (EMBED // 2)).astype(jnp.float32)
        freq = jnp.exp(-jnp.log(10000.0) * (2.0 * j) / EMBED)
        ang = pos * freq
        sincos = jnp.where(col < EMBED // 2, jnp.sin(ang), jnp.cos(ang))
        out_ref[...] = jnp.concatenate([st_ref[...], sincos], axis=-1)

    return pl.pallas_call(
        body,
        out_shape=jax.ShapeDtypeStruct((MAXSIZE, 2 * EMBED), jnp.float32),
    )(size_table)


def _make_sc_kernel():
    mesh = plsc.VectorSubcoreMesh(
        core_axis_name="c", subcore_axis_name="s",
        num_cores=NC, num_subcores=NS)

    @functools.partial(
        pl.kernel,
        out_type=jax.ShapeDtypeStruct((B, L, ROW), jnp.float32),
        mesh=mesh,
        scratch_types=[
            pltpu.VMEM((MAXSIZE, 2 * EMBED), jnp.float32),
            pltpu.VMEM((2 * TOKENS_PER_B,), jnp.int32),
            pltpu.VMEM((L, ROW), jnp.float32),
            pltpu.VMEM((L, ROW), jnp.float32),
            pltpu.SemaphoreType.DMA,
            pltpu.SemaphoreType.DMA,
        ],
        compiler_params=pltpu.CompilerParams(needs_layout_passes=False),
    )
    def sc_kernel(table_hbm, data_hbm, out_hbm,
                  table_v, idx_v, out_a, out_b, wsem_a, wsem_b):
        wid = lax.axis_index("s") * NC + lax.axis_index("c")
        b0 = wid * NB_PER_W
        pltpu.sync_copy(table_hbm, table_v)
        lane = lax.iota(jnp.int32, 16)
        zeros16 = jnp.zeros((16,), jnp.int32)

        def compute_block(idx_base, out_v):
            def body_l(l, carry):
                for c in range(NUM_CH):
                    p = idx_base + NUM_CH * l + c
                    rows16 = plsc.load_gather(idx_v, [zeros16 + p])
                    cbase = 0 if c == 0 else EMBED
                    for jj in range(EMBED // 16):
                        val = plsc.load_gather(
                            table_v, [rows16, cbase + jj * 16 + lane])
                        out_v[l, pl.ds(c * EMBED + jj * 16, 16)] = val
                return carry

            lax.fori_loop(0, L, body_l, 0)

        def wait_write(out_v, wsem):
            pltpu.make_async_copy(out_v, out_hbm.at[b0], wsem).wait()

        def body(m, carry):
            b = b0 + 2 * m
            pltpu.sync_copy(
                data_hbm.at[pl.ds(b * TOKENS_PER_B, 2 * TOKENS_PER_B)], idx_v)

            @pl.when(m > 0)
            def _():
                wait_write(out_a, wsem_a)
            compute_block(0, out_a)
            pltpu.async_copy(out_a, out_hbm.at[b], wsem_a)

            @pl.when(m > 0)
            def _():
                wait_write(out_b, wsem_b)
            compute_block(TOKENS_PER_B, out_b)
            pltpu.async_copy(out_b, out_hbm.at[b + 1], wsem_b)
            return carry

        lax.fori_loop(0, N_BPAIRS, body, 0)
        wait_write(out_a, wsem_a)
        wait_write(out_b, wsem_b)

    return sc_kernel


_sc_kernel = _make_sc_kernel()


def kernel(data, size_table):
    table = _build_table(size_table)
    data_flat = data.reshape(B * TOKENS_PER_B)
    return _sc_kernel(table, data_flat)


# speedup vs baseline: 3.7563x; 3.7563x over previous
import functools

import jax
import jax.numpy as jnp
from jax import lax
from jax.experimental import pallas as pl
from jax.experimental.pallas import tpu as pltpu
from jax.experimental.pallas import tpu_sc as plsc

EMBED = 64
MAXSIZE = 520
B = 4096
L = 50

NUM_CH = 6
ROW = NUM_CH * EMBED
VALS_PER_B = L * NUM_CH
NC, NS = 2, 16
NW = NC * NS
NB_PER_W = B // NW
GRP = 32
N_GRP = NB_PER_W // GRP
N_LPAIRS = L // 2


def _build_table(size_table):
    def body(st_ref, out_ref):
        pos = lax.broadcasted_iota(jnp.int32, (MAXSIZE, EMBED), 0).astype(jnp.float32)
        col = lax.broadcasted_iota(jnp.int32, (MAXSIZE, EMBED), 1)
        j = (col % (EMBED // 2)).astype(jnp.float32)
        freq = jnp.exp(-jnp.log(10000.0) * (2.0 * j) / EMBED)
        ang = pos * freq
        sincos = jnp.where(col < EMBED // 2, jnp.sin(ang), jnp.cos(ang))
        out_ref[...] = jnp.concatenate([st_ref[...], sincos], axis=-1)

    return pl.pallas_call(
        body,
        out_shape=jax.ShapeDtypeStruct((MAXSIZE, 2 * EMBED), jnp.float32),
    )(size_table)


def _make_sc_kernel():
    mesh = plsc.VectorSubcoreMesh(
        core_axis_name="c", subcore_axis_name="s",
        num_cores=NC, num_subcores=NS)

    @functools.partial(
        pl.kernel,
        out_type=jax.ShapeDtypeStruct((L, B, ROW), jnp.float32),
        mesh=mesh,
        scratch_types=[
            pltpu.VMEM((MAXSIZE, 2 * EMBED), jnp.float32),
            pltpu.VMEM((GRP * VALS_PER_B,), jnp.int32),
            pltpu.VMEM((GRP, ROW), jnp.float32),
            pltpu.VMEM((GRP, ROW), jnp.float32),
            pltpu.SemaphoreType.DMA,
            pltpu.SemaphoreType.DMA,
        ],
        compiler_params=pltpu.CompilerParams(needs_layout_passes=False),
    )
    def sc_kernel(table_hbm, data_hbm, out_hbm,
                  table_v, idx_v, out_a, out_b, wsem_a, wsem_b):
        wid = lax.axis_index("s") * NC + lax.axis_index("c")
        b0 = wid * NB_PER_W
        pltpu.sync_copy(table_hbm, table_v)
        lane = lax.iota(jnp.int32, 16)
        zeros16 = jnp.zeros((16,), jnp.int32)
        colvs = [
            (0 if c == 0 else EMBED) + jj * 16 + lane
            for c in range(NUM_CH) for jj in range(EMBED // 16)
        ]

        def assemble(l, out_v):
            @plsc.parallel_loop(0, GRP, unroll=4)
            def _(t):
                base = t * VALS_PER_B + NUM_CH * l
                for c in range(NUM_CH):
                    rows16 = plsc.load_gather(idx_v, [zeros16 + (base + c)])
                    for jj in range(EMBED // 16):
                        val = plsc.load_gather(
                            table_v, [rows16, colvs[c * 4 + jj]])
                        out_v[t, pl.ds(c * EMBED + jj * 16, 16)] = val

        def wait_write(out_v, wsem):
            pltpu.make_async_copy(
                out_v, out_hbm.at[0, pl.ds(b0, GRP)], wsem).wait()

        def body(gj, carry):
            g, j = gj // N_LPAIRS, gj % N_LPAIRS
            bq = b0 + g * GRP

            @pl.when(j == 0)
            def _():
                pltpu.sync_copy(
                    data_hbm.at[pl.ds(bq * VALS_PER_B, GRP * VALS_PER_B)],
                    idx_v)

            @pl.when(gj > 0)
            def _():
                wait_write(out_a, wsem_a)
            assemble(2 * j, out_a)
            pltpu.async_copy(out_a, out_hbm.at[2 * j, pl.ds(bq, GRP)], wsem_a)

            @pl.when(gj > 0)
            def _():
                wait_write(out_b, wsem_b)
            assemble(2 * j + 1, out_b)
            pltpu.async_copy(
                out_b, out_hbm.at[2 * j + 1, pl.ds(bq, GRP)], wsem_b)
            return carry

        lax.fori_loop(0, N_GRP * N_LPAIRS, body, 0)
        wait_write(out_a, wsem_a)
        wait_write(out_b, wsem_b)

    return sc_kernel


_sc_kernel = _make_sc_kernel()


def kernel(data, size_table):
    table = _build_table(size_table)
    data_flat = data.reshape(B * VALS_PER_B)
    out_lbf = _sc_kernel(table, data_flat)
    return jnp.transpose(out_lbf, (1, 0, 2))


# speedup vs baseline: 7.7032x; 2.0507x over previous
import functools

import jax
import jax.numpy as jnp
from jax import lax
from jax.experimental import pallas as pl
from jax.experimental.pallas import tpu as pltpu
from jax.experimental.pallas import tpu_sc as plsc

EMBED = 64
MAXSIZE = 520
B = 4096
L = 50

NUM_CH = 6
ROW = NUM_CH * EMBED
VALS_PER_B = L * NUM_CH
NC, NS = 2, 16
NW = NC * NS
NB_PER_W = B // NW
GRP = 32
N_GRP = NB_PER_W // GRP
N_LPAIRS = L // 2


def _build_table(size_table):
    def body(st_ref, out_ref):
        pos = lax.broadcasted_iota(jnp.int32, (MAXSIZE, EMBED), 0).astype(jnp.float32)
        col = lax.broadcasted_iota(jnp.int32, (MAXSIZE, EMBED), 1)
        j = (col % (EMBED // 2)).astype(jnp.float32)
        freq = jnp.exp(-jnp.log(10000.0) * (2.0 * j) / EMBED)
        ang = pos * freq
        sincos = jnp.where(col < EMBED // 2, jnp.sin(ang), jnp.cos(ang))
        out_ref[...] = jnp.concatenate([st_ref[...], sincos], axis=-1)

    return pl.pallas_call(
        body,
        out_shape=jax.ShapeDtypeStruct((MAXSIZE, 2 * EMBED), jnp.float32),
    )(size_table)


def _make_sc_kernel():
    mesh = plsc.VectorSubcoreMesh(
        core_axis_name="c", subcore_axis_name="s",
        num_cores=NC, num_subcores=NS)

    @functools.partial(
        pl.kernel,
        out_type=jax.ShapeDtypeStruct((L, B, ROW), jnp.float32),
        mesh=mesh,
        scratch_types=[
            pltpu.VMEM((MAXSIZE, 2 * EMBED), jnp.float32),
            pltpu.VMEM((VALS_PER_B, NB_PER_W), jnp.int32),
            pltpu.VMEM((GRP, ROW), jnp.float32),
            pltpu.VMEM((GRP, ROW), jnp.float32),
            pltpu.SemaphoreType.DMA,
            pltpu.SemaphoreType.DMA,
        ],
        compiler_params=pltpu.CompilerParams(needs_layout_passes=False),
    )
    def sc_kernel(table_hbm, data_hbm, out_hbm,
                  table_v, idx_v, out_a, out_b, wsem_a, wsem_b):
        wid = lax.axis_index("s") * NC + lax.axis_index("c")
        b0 = wid * NB_PER_W
        pltpu.sync_copy(table_hbm, table_v)
        pltpu.sync_copy(data_hbm.at[:, pl.ds(b0, NB_PER_W)], idx_v)
        lane = lax.iota(jnp.int32, 16)
        zeros16 = jnp.zeros((16,), jnp.int32)
        colvs = [
            (0 if c == 0 else EMBED) + jj * 16 + lane
            for c in range(NUM_CH) for jj in range(EMBED // 16)
        ]

        def assemble(l, bg, out_v):
            @plsc.parallel_loop(0, GRP, unroll=4)
            def _(t):
                bsplat = zeros16 + (bg + t)
                for c in range(NUM_CH):
                    rows16 = plsc.load_gather(
                        idx_v, [zeros16 + (c * L + l), bsplat])
                    for jj in range(EMBED // 16):
                        val = plsc.load_gather(
                            table_v, [rows16, colvs[c * 4 + jj]])
                        out_v[t, pl.ds(c * EMBED + jj * 16, 16)] = val

        def wait_write(out_v, wsem):
            pltpu.make_async_copy(
                out_v, out_hbm.at[0, pl.ds(b0, GRP)], wsem).wait()

        def body(gj, carry):
            g, j = gj // N_LPAIRS, gj % N_LPAIRS
            bg = g * GRP
            bq = b0 + bg

            @pl.when(gj > 0)
            def _():
                wait_write(out_a, wsem_a)
            assemble(2 * j, bg, out_a)
            pltpu.async_copy(out_a, out_hbm.at[2 * j, pl.ds(bq, GRP)], wsem_a)

            @pl.when(gj > 0)
            def _():
                wait_write(out_b, wsem_b)
            assemble(2 * j + 1, bg, out_b)
            pltpu.async_copy(
                out_b, out_hbm.at[2 * j + 1, pl.ds(bq, GRP)], wsem_b)
            return carry

        lax.fori_loop(0, N_GRP * N_LPAIRS, body, 0)
        wait_write(out_a, wsem_a)
        wait_write(out_b, wsem_b)

    return sc_kernel


_sc_kernel = _make_sc_kernel()


def kernel(data, size_table):
    table = _build_table(size_table)
    data_clb = jnp.transpose(data, (2, 1, 0)).reshape(VALS_PER_B, B)
    out_lbf = _sc_kernel(table, data_clb)
    return jnp.transpose(out_lbf, (1, 0, 2))
